# baseline (speedup 1.0000x reference)
import functools

import jax
import jax.numpy as jnp
from jax import lax
from jax.experimental import pallas as pl
from jax.experimental.pallas import tpu as pltpu
from jax.experimental.pallas import tpu_sc as plsc

BATCH = 4096
SEQ_LEN = 50
EMBED_DIM = 64
ROWS = BATCH * SEQ_LEN
NC, NS = 2, 16
NW = NC * NS
ROWS_PER_W = ROWS // NW
CHUNK = 128
NCHUNK = ROWS_PER_W // CHUNK


@functools.cache
def _make_gather():
    mesh = plsc.VectorSubcoreMesh(core_axis_name="c", subcore_axis_name="s",
                                  num_cores=NC, num_subcores=NS)

    @functools.partial(
        pl.kernel,
        mesh=mesh,
        out_type=jax.ShapeDtypeStruct((ROWS, EMBED_DIM), jnp.float32),
        scratch_types=[
            pltpu.VMEM((NCHUNK, CHUNK), jnp.int32),
            pltpu.VMEM((CHUNK, EMBED_DIM), jnp.float32),
            pltpu.VMEM((CHUNK, EMBED_DIM), jnp.float32),
            pltpu.SemaphoreType.DMA,
            pltpu.SemaphoreType.DMA,
        ],
        compiler_params=pltpu.CompilerParams(use_tc_tiling_on_sc=False),
    )
    def gather_k(ids_hbm, table_hbm, out_hbm, idx_v, buf0, buf1, sem0, sem1):
        wid = lax.axis_index("s") * NC + lax.axis_index("c")
        pltpu.sync_copy(ids_hbm.at[wid], idx_v)
        base = wid * ROWS_PER_W

        pltpu.async_copy(table_hbm.at[idx_v.at[0]], buf0, sem0)

        def body(j, _):
            def even():
                pltpu.make_async_copy(table_hbm.at[idx_v.at[0]], buf0,
                                      sem0).wait()

                @pl.when(j < NCHUNK - 1)
                def _():
                    pltpu.async_copy(table_hbm.at[idx_v.at[j + 1]], buf1,
                                     sem1)

                pltpu.sync_copy(buf0,
                                out_hbm.at[pl.ds(base + j * CHUNK, CHUNK)])

            def odd():
                pltpu.make_async_copy(table_hbm.at[idx_v.at[0]], buf1,
                                      sem1).wait()

                @pl.when(j < NCHUNK - 1)
                def _():
                    pltpu.async_copy(table_hbm.at[idx_v.at[j + 1]], buf0,
                                     sem0)

                pltpu.sync_copy(buf1,
                                out_hbm.at[pl.ds(base + j * CHUNK, CHUNK)])

            pl.when((j & 1) == 0)(even)
            pl.when((j & 1) == 1)(odd)
            return 0

        lax.fori_loop(0, NCHUNK, body, 0)

    return gather_k


_BLK = 2048


def _mm_body(ue_ref, soc_ref, w1_ref, w2_ref, b_ref, out_ref):
    acc = jnp.dot(ue_ref[...], w1_ref[...], preferred_element_type=jnp.float32)
    acc += jnp.dot(soc_ref[...], w2_ref[...], preferred_element_type=jnp.float32)
    out_ref[...] = jnp.maximum(acc + b_ref[...], 0.0)


def _fused_linear(ue, soc, w1t, w2t, b2d):
    return pl.pallas_call(
        _mm_body,
        grid=(ROWS // _BLK,),
        in_specs=[
            pl.BlockSpec((_BLK, EMBED_DIM), lambda i: (i, 0)),
            pl.BlockSpec((_BLK, EMBED_DIM), lambda i: (i, 0)),
            pl.BlockSpec((EMBED_DIM, EMBED_DIM), lambda i: (0, 0)),
            pl.BlockSpec((EMBED_DIM, EMBED_DIM), lambda i: (0, 0)),
            pl.BlockSpec((1, EMBED_DIM), lambda i: (0, 0)),
        ],
        out_specs=pl.BlockSpec((_BLK, EMBED_DIM), lambda i: (i, 0)),
        out_shape=jax.ShapeDtypeStruct((ROWS, EMBED_DIM), jnp.float32),
        compiler_params=pltpu.CompilerParams(
            dimension_semantics=("arbitrary",)),
    )(ue, soc, w1t, w2t, b2d)


def kernel(user_embeds, user_ids, emb_table, W, b):
    ids = user_ids.astype(jnp.int32).reshape(NW, NCHUNK, CHUNK)
    social = _make_gather()(ids, emb_table)
    ue = user_embeds.reshape(ROWS, EMBED_DIM)
    wt = W.T
    out = _fused_linear(ue, social, wt[:EMBED_DIM], wt[EMBED_DIM:],
                        b.reshape(1, EMBED_DIM))
    return out.reshape(BATCH, SEQ_LEN, EMBED_DIM)


# speedup vs baseline: 1.6035x; 1.6035x over previous
import functools

import jax
import jax.numpy as jnp
from jax import lax
from jax.experimental import pallas as pl
from jax.experimental.pallas import tpu as pltpu
from jax.experimental.pallas import tpu_sc as plsc

BATCH = 4096
SEQ_LEN = 50
EMBED_DIM = 64
ROWS = BATCH * SEQ_LEN
NTILE = 125000
NC, NS = 2, 16
NW = NC * NS
ROWS_PER_W = ROWS // NW
CHUNK = 128
NCHUNK = ROWS_PER_W // CHUNK


@functools.cache
def _make_gather():
    mesh = plsc.VectorSubcoreMesh(core_axis_name="c", subcore_axis_name="s",
                                  num_cores=NC, num_subcores=NS)

    @functools.partial(
        pl.kernel,
        mesh=mesh,
        out_type=jax.ShapeDtypeStruct((ROWS, EMBED_DIM), jnp.float32),
        scratch_types=[
            pltpu.VMEM((NCHUNK, CHUNK), jnp.int32),
            pltpu.VMEM((CHUNK, EMBED_DIM), jnp.float32),
            pltpu.VMEM((CHUNK, EMBED_DIM), jnp.float32),
            pltpu.SemaphoreType.DMA,
            pltpu.SemaphoreType.DMA,
        ],
        compiler_params=pltpu.CompilerParams(use_tc_tiling_on_sc=True),
    )
    def gather_k(ids_hbm, table_hbm, out_hbm, idx_v, buf0, buf1, sem0, sem1):
        wid = lax.axis_index("s") * NC + lax.axis_index("c")
        pltpu.sync_copy(ids_hbm.at[wid], idx_v)
        base = wid * ROWS_PER_W

        def fetch_chunk(j, buf, sem):
            def grp_body(g, _):
                vids = idx_v[j, pl.ds(g * 16, 16)]
                for k in range(16):
                    rid = vids[k]
                    pltpu.async_copy(table_hbm.at[rid >> 3, rid & 7],
                                     buf.at[g * 16 + k], sem)
                return 0

            lax.fori_loop(0, CHUNK // 16, grp_body, 0)

        def drain_chunk(buf, sem):
            def drain_body(r, _):
                pltpu.make_async_copy(table_hbm.at[0, 0], buf.at[r],
                                      sem).wait()
                return 0

            lax.fori_loop(0, CHUNK, drain_body, 0)

        fetch_chunk(0, buf0, sem0)

        def body(j, _):
            def even():
                drain_chunk(buf0, sem0)

                @pl.when(j < NCHUNK - 1)
                def _():
                    fetch_chunk(j + 1, buf1, sem1)

                pltpu.sync_copy(buf0,
                                out_hbm.at[pl.ds(base + j * CHUNK, CHUNK)])

            def odd():
                drain_chunk(buf1, sem1)

                @pl.when(j < NCHUNK - 1)
                def _():
                    fetch_chunk(j + 1, buf0, sem0)

                pltpu.sync_copy(buf1,
                                out_hbm.at[pl.ds(base + j * CHUNK, CHUNK)])

            pl.when((j & 1) == 0)(even)
            pl.when((j & 1) == 1)(odd)
            return 0

        lax.fori_loop(0, NCHUNK, body, 0)

    return gather_k


_BLK = 2048


def _mm_body(ue_ref, soc_ref, w1_ref, w2_ref, b_ref, out_ref):
    acc = jnp.dot(ue_ref[...], w1_ref[...], preferred_element_type=jnp.float32)
    acc += jnp.dot(soc_ref[...], w2_ref[...], preferred_element_type=jnp.float32)
    out_ref[...] = jnp.maximum(acc + b_ref[...], 0.0)


def _fused_linear(ue, soc, w1t, w2t, b2d):
    return pl.pallas_call(
        _mm_body,
        grid=(ROWS // _BLK,),
        in_specs=[
            pl.BlockSpec((_BLK, EMBED_DIM), lambda i: (i, 0)),
            pl.BlockSpec((_BLK, EMBED_DIM), lambda i: (i, 0)),
            pl.BlockSpec((EMBED_DIM, EMBED_DIM), lambda i: (0, 0)),
            pl.BlockSpec((EMBED_DIM, EMBED_DIM), lambda i: (0, 0)),
            pl.BlockSpec((1, EMBED_DIM), lambda i: (0, 0)),
        ],
        out_specs=pl.BlockSpec((_BLK, EMBED_DIM), lambda i: (i, 0)),
        out_shape=jax.ShapeDtypeStruct((ROWS, EMBED_DIM), jnp.float32),
        compiler_params=pltpu.CompilerParams(
            dimension_semantics=("arbitrary",)),
    )(ue, soc, w1t, w2t, b2d)


def kernel(user_embeds, user_ids, emb_table, W, b):
    ids = user_ids.astype(jnp.int32).reshape(NW, NCHUNK, CHUNK)
    table3 = emb_table.reshape(NTILE, 8, EMBED_DIM)
    social = _make_gather()(ids, table3)
    ue = user_embeds.reshape(ROWS, EMBED_DIM)
    wt = W.T
    out = _fused_linear(ue, social, wt[:EMBED_DIM], wt[EMBED_DIM:],
                        b.reshape(1, EMBED_DIM))
    return out.reshape(BATCH, SEQ_LEN, EMBED_DIM)


# speedup vs baseline: 1.8760x; 1.1699x over previous
import functools

import jax
import jax.numpy as jnp
from jax import lax
from jax.experimental import pallas as pl
from jax.experimental.pallas import tpu as pltpu
from jax.experimental.pallas import tpu_sc as plsc

BATCH = 4096
SEQ_LEN = 50
EMBED_DIM = 64
ROWS = BATCH * SEQ_LEN
NTILE = 125000
NC, NS = 2, 16
NW = NC * NS
ROWS_PER_W = ROWS // NW
CHUNK = 128
NCHUNK = ROWS_PER_W // CHUNK


@functools.cache
def _make_gather():
    mesh = plsc.VectorSubcoreMesh(core_axis_name="c", subcore_axis_name="s",
                                  num_cores=NC, num_subcores=NS)

    @functools.partial(
        pl.kernel,
        mesh=mesh,
        out_type=jax.ShapeDtypeStruct((ROWS // 2, 2 * EMBED_DIM),
                                      jnp.float32),
        scratch_types=[
            pltpu.VMEM((NCHUNK, CHUNK), jnp.int32),
            pltpu.VMEM((CHUNK // 2, 2 * EMBED_DIM), jnp.float32),
            pltpu.VMEM((CHUNK // 2, 2 * EMBED_DIM), jnp.float32),
            pltpu.SemaphoreType.DMA,
            pltpu.SemaphoreType.DMA,
        ],
        compiler_params=pltpu.CompilerParams(use_tc_tiling_on_sc=True),
    )
    def gather_k(ids_hbm, table_hbm, out_hbm, idx_v, buf0, buf1, sem0, sem1):
        wid = lax.axis_index("s") * NC + lax.axis_index("c")
        pltpu.sync_copy(ids_hbm.at[wid], idx_v)
        base = wid * (ROWS_PER_W // 2)

        def fetch_chunk(j, buf, sem):
            def grp_body(g, _):
                vids = idx_v[j, pl.ds(g * 16, 16)]
                for k in range(16):
                    rid = vids[k]
                    pltpu.async_copy(
                        table_hbm.at[rid >> 3, rid & 7],
                        buf.at[g * 8 + k // 2,
                               pl.ds((k % 2) * EMBED_DIM, EMBED_DIM)],
                        sem)
                return 0

            lax.fori_loop(0, CHUNK // 16, grp_body, 0)

        def drain_chunk(buf, sem):
            def drain_body(r, _):
                pltpu.make_async_copy(
                    table_hbm.at[0, 0],
                    buf.at[0, pl.ds(0, EMBED_DIM)], sem).wait()
                return 0

            lax.fori_loop(0, CHUNK, drain_body, 0)

        fetch_chunk(0, buf0, sem0)

        def body(j, _):
            def even():
                drain_chunk(buf0, sem0)

                @pl.when(j < NCHUNK - 1)
                def _():
                    fetch_chunk(j + 1, buf1, sem1)

                pltpu.sync_copy(
                    buf0,
                    out_hbm.at[pl.ds(base + j * (CHUNK // 2), CHUNK // 2)])

            def odd():
                drain_chunk(buf1, sem1)

                @pl.when(j < NCHUNK - 1)
                def _():
                    fetch_chunk(j + 1, buf0, sem0)

                pltpu.sync_copy(
                    buf1,
                    out_hbm.at[pl.ds(base + j * (CHUNK // 2), CHUNK // 2)])

            pl.when((j & 1) == 0)(even)
            pl.when((j & 1) == 1)(odd)
            return 0

        lax.fori_loop(0, NCHUNK, body, 0)

    return gather_k


_BLK = 1024
PAIR_ROWS = ROWS // 2
PD = 2 * EMBED_DIM


def _mm_body(ue_ref, soc_ref, w1_ref, w2_ref, b_ref, out_ref):
    acc = jnp.dot(ue_ref[...], w1_ref[...], preferred_element_type=jnp.float32)
    acc += jnp.dot(soc_ref[...], w2_ref[...], preferred_element_type=jnp.float32)
    out_ref[...] = jnp.maximum(acc + b_ref[...], 0.0)


def _fused_linear(ue_p, soc_p, w1bd, w2bd, b_p):
    return pl.pallas_call(
        _mm_body,
        grid=(PAIR_ROWS // _BLK,),
        in_specs=[
            pl.BlockSpec((_BLK, PD), lambda i: (i, 0)),
            pl.BlockSpec((_BLK, PD), lambda i: (i, 0)),
            pl.BlockSpec((PD, PD), lambda i: (0, 0)),
            pl.BlockSpec((PD, PD), lambda i: (0, 0)),
            pl.BlockSpec((1, PD), lambda i: (0, 0)),
        ],
        out_specs=pl.BlockSpec((_BLK, PD), lambda i: (i, 0)),
        out_shape=jax.ShapeDtypeStruct((PAIR_ROWS, PD), jnp.float32),
        compiler_params=pltpu.CompilerParams(
            dimension_semantics=("arbitrary",)),
    )(ue_p, soc_p, w1bd, w2bd, b_p)


def _blockdiag(m):
    z = jnp.zeros((EMBED_DIM, EMBED_DIM), m.dtype)
    return jnp.block([[m, z], [z, m]])


def kernel(user_embeds, user_ids, emb_table, W, b):
    ids = user_ids.astype(jnp.int32).reshape(NW, NCHUNK, CHUNK)
    table3 = emb_table.reshape(NTILE, 8, EMBED_DIM)
    social_p = _make_gather()(ids, table3)
    ue_p = user_embeds.reshape(PAIR_ROWS, PD)
    wt = W.T
    w1bd = _blockdiag(wt[:EMBED_DIM])
    w2bd = _blockdiag(wt[EMBED_DIM:])
    b_p = jnp.concatenate([b, b]).reshape(1, PD)
    out = _fused_linear(ue_p, social_p, w1bd, w2bd, b_p)
    return out.reshape(BATCH, SEQ_LEN, EMBED_DIM)
